# baseline (device time: 106703 ns/iter reference)
import jax
import jax.numpy as jnp
from jax import lax
from jax.experimental import pallas as pl
from jax.experimental.pallas import tpu as pltpu

M = 4096
N = 2048
HALF_M = M // 2
HALF_N = N // 2
C = 64
CH = HALF_M // C


def kernel(x):
    def body(x_ref, out_ref, xrecv_ref, lbuf_ref,
             sem_sx, sem_rx, sem_l, sem_sy, sem_ry):
        me_x = lax.axis_index("x")
        me_y = lax.axis_index("y")
        x_peer = (1 - me_x, me_y)
        y_peer = (me_x, 1 - me_y)
        pcol = (1 - me_x) * HALF_N
        mcol = me_x * HALF_N
        r0 = me_y * HALF_M

        barrier = pltpu.get_barrier_semaphore()
        for nbr in (x_peer, y_peer):
            pl.semaphore_signal(barrier, inc=1, device_id=nbr,
                                device_id_type=pl.DeviceIdType.MESH)
        pl.semaphore_wait(barrier, 2)

        def x_rdma(k):
            return pltpu.make_async_remote_copy(
                src_ref=x_ref.at[0, pl.ds(r0 + k * CH, CH), pl.ds(pcol, HALF_N)],
                dst_ref=xrecv_ref.at[pl.ds(k * CH, CH), :],
                send_sem=sem_sx.at[k],
                recv_sem=sem_rx.at[k],
                device_id=x_peer,
                device_id_type=pl.DeviceIdType.MESH,
            )

        def l_copy(k):
            return pltpu.make_async_copy(
                x_ref.at[0, pl.ds(r0 + k * CH, CH), pl.ds(mcol, HALF_N)],
                lbuf_ref.at[pl.ds(k * CH, CH), :],
                sem_l.at[k],
            )

        def y_rdma(k):
            return pltpu.make_async_remote_copy(
                src_ref=out_ref.at[pl.ds(r0 + k * CH, CH), :],
                dst_ref=out_ref.at[pl.ds(r0 + k * CH, CH), :],
                send_sem=sem_sy.at[k],
                recv_sem=sem_ry.at[k],
                device_id=y_peer,
                device_id_type=pl.DeviceIdType.MESH,
            )

        for k in range(C):
            x_rdma(k).start()
            l_copy(k).start()

        for k in range(C):
            l_copy(k).wait()
            x_rdma(k).wait_recv()
            out_ref[pl.ds(r0 + k * CH, CH), :] = (
                xrecv_ref[pl.ds(k * CH, CH), :]
                + lbuf_ref[pl.ds(k * CH, CH), :]
            )
            y_rdma(k).start()

        for k in range(C):
            x_rdma(k).wait_send()
            y_rdma(k).wait_send()
            y_rdma(k).wait_recv()

    return pl.pallas_call(
        body,
        out_shape=jax.ShapeDtypeStruct((M, HALF_N), jnp.float32),
        in_specs=[pl.BlockSpec(memory_space=pl.ANY)],
        out_specs=pl.BlockSpec(memory_space=pltpu.VMEM),
        scratch_shapes=[
            pltpu.VMEM((HALF_M, HALF_N), jnp.float32),
            pltpu.VMEM((HALF_M, HALF_N), jnp.float32),
            pltpu.SemaphoreType.DMA((C,)),
            pltpu.SemaphoreType.DMA((C,)),
            pltpu.SemaphoreType.DMA((C,)),
            pltpu.SemaphoreType.DMA((C,)),
            pltpu.SemaphoreType.DMA((C,)),
        ],
        compiler_params=pltpu.CompilerParams(collective_id=0),
    )(x)


# device time: 106635 ns/iter; 1.0006x vs baseline; 1.0006x over previous
import jax
import jax.numpy as jnp
from jax import lax
from jax.experimental import pallas as pl
from jax.experimental.pallas import tpu as pltpu

M = 4096
N = 2048
HALF_M = M // 2
HALF_N = N // 2
C = 32
CH = HALF_M // C


def kernel(x):
    def body(x_ref, out_ref, xrecv_ref, lbuf_ref,
             sem_sx, sem_rx, sem_l, sem_sy, sem_ry):
        me_x = lax.axis_index("x")
        me_y = lax.axis_index("y")
        x_peer = (1 - me_x, me_y)
        y_peer = (me_x, 1 - me_y)
        pcol = (1 - me_x) * HALF_N
        mcol = me_x * HALF_N
        r0 = me_y * HALF_M

        barrier = pltpu.get_barrier_semaphore()
        for nbr in (x_peer, y_peer):
            pl.semaphore_signal(barrier, inc=1, device_id=nbr,
                                device_id_type=pl.DeviceIdType.MESH)
        pl.semaphore_wait(barrier, 2)

        def x_rdma(k):
            return pltpu.make_async_remote_copy(
                src_ref=x_ref.at[0, pl.ds(r0 + k * CH, CH), pl.ds(pcol, HALF_N)],
                dst_ref=xrecv_ref.at[pl.ds(k * CH, CH), :],
                send_sem=sem_sx.at[k],
                recv_sem=sem_rx.at[k],
                device_id=x_peer,
                device_id_type=pl.DeviceIdType.MESH,
            )

        def l_copy(k):
            return pltpu.make_async_copy(
                x_ref.at[0, pl.ds(r0 + k * CH, CH), pl.ds(mcol, HALF_N)],
                lbuf_ref.at[pl.ds(k * CH, CH), :],
                sem_l.at[k],
            )

        def y_rdma(k):
            return pltpu.make_async_remote_copy(
                src_ref=out_ref.at[pl.ds(r0 + k * CH, CH), :],
                dst_ref=out_ref.at[pl.ds(r0 + k * CH, CH), :],
                send_sem=sem_sy.at[k],
                recv_sem=sem_ry.at[k],
                device_id=y_peer,
                device_id_type=pl.DeviceIdType.MESH,
            )

        for k in range(C):
            x_rdma(k).start()
            l_copy(k).start()

        for k in range(C):
            l_copy(k).wait()
            x_rdma(k).wait_recv()
            out_ref[pl.ds(r0 + k * CH, CH), :] = (
                xrecv_ref[pl.ds(k * CH, CH), :]
                + lbuf_ref[pl.ds(k * CH, CH), :]
            )
            y_rdma(k).start()

        for k in range(C):
            x_rdma(k).wait_send()
            y_rdma(k).wait_send()
            y_rdma(k).wait_recv()

    return pl.pallas_call(
        body,
        out_shape=jax.ShapeDtypeStruct((M, HALF_N), jnp.float32),
        in_specs=[pl.BlockSpec(memory_space=pl.ANY)],
        out_specs=pl.BlockSpec(memory_space=pltpu.VMEM),
        scratch_shapes=[
            pltpu.VMEM((HALF_M, HALF_N), jnp.float32),
            pltpu.VMEM((HALF_M, HALF_N), jnp.float32),
            pltpu.SemaphoreType.DMA((C,)),
            pltpu.SemaphoreType.DMA((C,)),
            pltpu.SemaphoreType.DMA((C,)),
            pltpu.SemaphoreType.DMA((C,)),
            pltpu.SemaphoreType.DMA((C,)),
        ],
        compiler_params=pltpu.CompilerParams(collective_id=0),
    )(x)
